# Initial kernel scaffold; baseline (speedup 1.0000x reference)
import functools

import jax
import jax.numpy as jnp
from jax import lax
from jax.experimental import pallas as pl
from jax.experimental.pallas import tpu as pltpu
from jax.experimental.pallas import tpu_sc as plsc

NC = 2
NS = 16
K = 128


def _sc_spmm(h, src, dst, w, n_chunks):
    N, D = h.shape
    rows_per_tile = N // NS
    zrows = 125
    mesh = plsc.VectorSubcoreMesh(core_axis_name="c", subcore_axis_name="s")

    @functools.partial(
        pl.kernel,
        mesh=mesh,
        out_type=jax.ShapeDtypeStruct((NC, N, D), jnp.float32),
        scratch_types=[
            pltpu.VMEM((K,), jnp.int32),
            pltpu.VMEM((K,), jnp.int32),
            pltpu.VMEM((K,), jnp.float32),
            pltpu.VMEM((K, D), jnp.float32),
            pltpu.VMEM((zrows, D), jnp.float32),
            pltpu.VMEM_SHARED((N, D), jnp.float32),
        ],
    )
    def spmm(h_hbm, src_hbm, dst_hbm, w_hbm, out_hbm,
             src_v, dst_v, w_v, rows_v, zbuf, acc):
        c = lax.axis_index("c")
        s = lax.axis_index("s")
        wid = c * NS + s

        def zrow(i, carry):
            for r in range(D // 16):
                zbuf[i, pl.ds(r * 16, 16)] = jnp.zeros((16,), jnp.float32)
            return carry
        lax.fori_loop(0, zrows, zrow, 0)
        for b in range(rows_per_tile // zrows):
            pltpu.sync_copy(
                zbuf, acc.at[pl.ds(s * rows_per_tile + b * zrows, zrows)])
        plsc.subcore_barrier()

        base0 = wid * (n_chunks * K)

        def body(ci, carry):
            base = base0 + ci * K
            pltpu.sync_copy(src_hbm.at[pl.ds(base, K)], src_v)
            pltpu.sync_copy(dst_hbm.at[pl.ds(base, K)], dst_v)
            pltpu.sync_copy(w_hbm.at[pl.ds(base, K)], w_v)
            pltpu.sync_copy(h_hbm.at[src_v], rows_v)

            def scale(e, carry2):
                we = w_v[e]
                for r in range(D // 16):
                    sl = pl.ds(r * 16, 16)
                    rows_v[e, sl] = rows_v[e, sl] * we
                return carry2
            lax.fori_loop(0, K, scale, 0)
            pltpu.sync_copy(rows_v, acc.at[dst_v], add=True)
            return carry
        lax.fori_loop(0, n_chunks, body, 0)

        plsc.subcore_barrier()
        sl = pl.ds(s * rows_per_tile, rows_per_tile)
        pltpu.sync_copy(acc.at[sl], out_hbm.at[c, sl])

    return spmm(h, src, dst, w)


def _tc_mm(x, W):
    N, F = x.shape
    H = W.shape[1]
    BN = 1000

    def body(x_ref, w_ref, o_ref):
        o_ref[...] = jnp.dot(x_ref[...], w_ref[...],
                             preferred_element_type=jnp.float32)

    return pl.pallas_call(
        body,
        grid=(N // BN,),
        in_specs=[pl.BlockSpec((BN, F), lambda i: (i, 0)),
                  pl.BlockSpec((F, H), lambda i: (0, 0))],
        out_specs=pl.BlockSpec((BN, H), lambda i: (i, 0)),
        out_shape=jax.ShapeDtypeStruct((N, H), jnp.float32),
    )(x, W)


def _tc_fuse(p0, p1, b, W):
    N, F = p0.shape
    H = W.shape[1]
    BN = 1000

    def body(p0_ref, p1_ref, b_ref, w_ref, o_ref):
        hblk = jnp.maximum(p0_ref[...] + p1_ref[...] + b_ref[...], 0.0)
        o_ref[...] = jnp.dot(hblk, w_ref[...],
                             preferred_element_type=jnp.float32)

    return pl.pallas_call(
        body,
        grid=(N // BN,),
        in_specs=[pl.BlockSpec((BN, F), lambda i: (i, 0)),
                  pl.BlockSpec((BN, F), lambda i: (i, 0)),
                  pl.BlockSpec((1, F), lambda i: (0, 0)),
                  pl.BlockSpec((F, H), lambda i: (0, 0))],
        out_specs=pl.BlockSpec((BN, H), lambda i: (i, 0)),
        out_shape=jax.ShapeDtypeStruct((N, H), jnp.float32),
    )(p0, p1, b, W)


def _tc_add(q0, q1, b):
    N, F = q0.shape
    BN = 1000

    def body(q0_ref, q1_ref, b_ref, o_ref):
        o_ref[...] = q0_ref[...] + q1_ref[...] + b_ref[...]

    return pl.pallas_call(
        body,
        grid=(N // BN,),
        in_specs=[pl.BlockSpec((BN, F), lambda i: (i, 0)),
                  pl.BlockSpec((BN, F), lambda i: (i, 0)),
                  pl.BlockSpec((1, F), lambda i: (0, 0))],
        out_specs=pl.BlockSpec((BN, F), lambda i: (i, 0)),
        out_shape=jax.ShapeDtypeStruct((N, F), jnp.float32),
    )(q0, q1, b)


def kernel(x, edge_index, edge_weight, W1, b1, W2, b2):
    N, F = x.shape
    E = edge_weight.shape[0]
    src = edge_index[0]
    dst = edge_index[1]

    sweep = NC * NS * K
    n_chunks = -(-E // sweep)
    pad = n_chunks * sweep - E
    src_p = jnp.pad(src, (0, pad))
    dst_p = jnp.pad(dst, (0, pad))
    w_p = jnp.pad(edge_weight, (0, pad))

    h = _tc_mm(x, W1)
    p = _sc_spmm(h, src_p, dst_p, w_p, n_chunks)

    C = W2.shape[1]
    Cp = 64
    W2p = jnp.pad(W2, ((0, 0), (0, Cp - C)))
    h2 = _tc_fuse(p[0], p[1], b1.reshape(1, -1), W2p)

    q = _sc_spmm(h2, src_p, dst_p, w_p, n_chunks)
    outp = _tc_add(q[0], q[1], jnp.pad(b2, (0, Cp - C)).reshape(1, -1))
    return outp[:, :C]


# speedup vs baseline: 2.3843x; 2.3843x over previous
import functools

import jax
import jax.numpy as jnp
from jax import lax
from jax.experimental import pallas as pl
from jax.experimental.pallas import tpu as pltpu
from jax.experimental.pallas import tpu_sc as plsc

NC = 2
NS = 16
K = 128


NBUF = 2
ZR = 32


def _sc_spmm(h, src, dst, w, n_chunks):
    N, D = h.shape
    NP = ((N + NS * ZR - 1) // (NS * ZR)) * NS * ZR
    RPT = NP // NS
    assert n_chunks % NBUF == 0 and RPT % ZR == 0
    mesh = plsc.VectorSubcoreMesh(core_axis_name="c", subcore_axis_name="s")

    @functools.partial(
        pl.kernel,
        mesh=mesh,
        out_type=jax.ShapeDtypeStruct((NC, NP, D), jnp.float32),
        scratch_types=[
            pltpu.VMEM((NBUF, K), jnp.int32),
            pltpu.VMEM((NBUF, K), jnp.int32),
            pltpu.VMEM((NBUF, K), jnp.float32),
            pltpu.VMEM((NBUF, K, D), jnp.float32),
            pltpu.VMEM((ZR, D), jnp.float32),
            pltpu.VMEM_SHARED((NP, D), jnp.float32),
        ],
    )
    def spmm(h_hbm, src_hbm, dst_hbm, w_hbm, out_hbm,
             src_v, dst_v, w_v, rows_v, zbuf, acc):
        c = lax.axis_index("c")
        s = lax.axis_index("s")
        wid = c * NS + s

        def zrow(i, carry):
            for r in range(D // 16):
                zbuf[i, pl.ds(r * 16, 16)] = jnp.zeros((16,), jnp.float32)
            return carry
        lax.fori_loop(0, ZR, zrow, 0)
        for b in range(RPT // ZR):
            pltpu.sync_copy(zbuf, acc.at[pl.ds(s * RPT + b * ZR, ZR)])
        plsc.subcore_barrier()

        base0 = wid * (n_chunks * K)

        def outer(oi, carry):
            for j in range(NBUF):
                base = base0 + (oi * NBUF + j) * K
                pltpu.sync_copy(src_hbm.at[pl.ds(base, K)], src_v.at[j])
                pltpu.sync_copy(dst_hbm.at[pl.ds(base, K)], dst_v.at[j])
                pltpu.sync_copy(w_hbm.at[pl.ds(base, K)], w_v.at[j])
                pltpu.sync_copy(h_hbm.at[src_v.at[j]], rows_v.at[j])

                def scale(g, carry2):
                    wv = w_v[j, pl.ds(g * 16, 16)]
                    for l in range(16):
                        we = wv[l]
                        e = g * 16 + l
                        for r in range(D // 16):
                            sl = pl.ds(r * 16, 16)
                            rows_v[j, e, sl] = rows_v[j, e, sl] * we
                    return carry2
                lax.fori_loop(0, K // 16, scale, 0)
                pltpu.sync_copy(rows_v.at[j], acc.at[dst_v.at[j]], add=True)
                plsc.subcore_barrier()
            return carry
        lax.fori_loop(0, n_chunks // NBUF, outer, 0)

        plsc.subcore_barrier()
        sl = pl.ds(s * RPT, RPT)
        pltpu.sync_copy(acc.at[sl], out_hbm.at[c, sl])

    return spmm(h, src, dst, w)


def _tc_mm(x, W):
    N, F = x.shape
    H = W.shape[1]
    BN = 1000

    def body(x_ref, w_ref, o_ref):
        o_ref[...] = jnp.dot(x_ref[...], w_ref[...],
                             preferred_element_type=jnp.float32)

    return pl.pallas_call(
        body,
        grid=(N // BN,),
        in_specs=[pl.BlockSpec((BN, F), lambda i: (i, 0)),
                  pl.BlockSpec((F, H), lambda i: (0, 0))],
        out_specs=pl.BlockSpec((BN, H), lambda i: (i, 0)),
        out_shape=jax.ShapeDtypeStruct((N, H), jnp.float32),
    )(x, W)


def _tc_relu(p0, p1, b):
    N, F = p0.shape
    BN = 1000

    def body(p0_ref, p1_ref, b_ref, o_ref):
        o_ref[...] = jnp.maximum(p0_ref[...] + p1_ref[...] + b_ref[...], 0.0)

    return pl.pallas_call(
        body,
        grid=(N // BN,),
        in_specs=[pl.BlockSpec((BN, F), lambda i: (i, 0)),
                  pl.BlockSpec((BN, F), lambda i: (i, 0)),
                  pl.BlockSpec((1, F), lambda i: (0, 0))],
        out_specs=pl.BlockSpec((BN, F), lambda i: (i, 0)),
        out_shape=jax.ShapeDtypeStruct((N, F), jnp.float32),
    )(p0, p1, b)


def _tc_out(q0, q1, W, b):
    N, F = q0.shape
    C = W.shape[1]
    BN = 1000

    def body(q0_ref, q1_ref, w_ref, b_ref, o_ref):
        o_ref[...] = jnp.dot(q0_ref[...] + q1_ref[...], w_ref[...],
                             preferred_element_type=jnp.float32) + b_ref[...]

    return pl.pallas_call(
        body,
        grid=(N // BN,),
        in_specs=[pl.BlockSpec((BN, F), lambda i: (i, 0)),
                  pl.BlockSpec((BN, F), lambda i: (i, 0)),
                  pl.BlockSpec((F, C), lambda i: (0, 0)),
                  pl.BlockSpec((1, C), lambda i: (0, 0))],
        out_specs=pl.BlockSpec((BN, C), lambda i: (i, 0)),
        out_shape=jax.ShapeDtypeStruct((N, C), jnp.float32),
    )(q0, q1, W, b)


def kernel(x, edge_index, edge_weight, W1, b1, W2, b2):
    N, F = x.shape
    E = edge_weight.shape[0]
    src = edge_index[0]
    dst = edge_index[1]

    sweep = NC * NS * K
    n_chunks = -(-E // sweep)
    n_chunks += (-n_chunks) % NBUF
    pad = n_chunks * sweep - E
    src_p = jnp.pad(src, (0, pad))
    dst_p = jnp.pad(dst, (0, pad))
    w_p = jnp.pad(edge_weight, (0, pad))

    h = _tc_mm(x, W1)
    p = _sc_spmm(h, src_p, dst_p, w_p, n_chunks)
    h2 = _tc_relu(p[0, :N], p[1, :N], b1.reshape(1, -1))
    q = _sc_spmm(h2, src_p, dst_p, w_p, n_chunks)
    return _tc_out(q[0, :N], q[1, :N], W2, b2.reshape(1, -1))


# speedup vs baseline: 2.4442x; 1.0251x over previous
import functools

import jax
import jax.numpy as jnp
from jax import lax
from jax.experimental import pallas as pl
from jax.experimental.pallas import tpu as pltpu
from jax.experimental.pallas import tpu_sc as plsc

NC = 2
NS = 16
K = 128


NBUF = 2
ZR = 32


def _sc_spmm(h, src, dst, w, n_chunks):
    N, D = h.shape
    NP = ((N + NS * ZR - 1) // (NS * ZR)) * NS * ZR
    RPT = NP // NS
    assert n_chunks % NBUF == 0 and RPT % ZR == 0
    mesh = plsc.VectorSubcoreMesh(core_axis_name="c", subcore_axis_name="s")

    @functools.partial(
        pl.kernel,
        mesh=mesh,
        out_type=jax.ShapeDtypeStruct((NC, NP, D), jnp.float32),
        scratch_types=[
            pltpu.VMEM((NBUF, K), jnp.int32),
            pltpu.VMEM((NBUF, K), jnp.int32),
            pltpu.VMEM((NBUF, K), jnp.float32),
            pltpu.VMEM((NBUF, K, D), jnp.float32),
            pltpu.VMEM((ZR, D), jnp.float32),
            pltpu.VMEM_SHARED((NP, D), jnp.float32),
        ],
    )
    def spmm(h_hbm, src_hbm, dst_hbm, w_hbm, out_hbm,
             src_v, dst_v, w_v, rows_v, zbuf, acc):
        c = lax.axis_index("c")
        s = lax.axis_index("s")
        wid = c * NS + s

        def zrow(i, carry):
            for r in range(D // 16):
                zbuf[i, pl.ds(r * 16, 16)] = jnp.zeros((16,), jnp.float32)
            return carry
        lax.fori_loop(0, ZR, zrow, 0)
        for b in range(RPT // ZR):
            pltpu.sync_copy(zbuf, acc.at[pl.ds(s * RPT + b * ZR, ZR)])
        plsc.subcore_barrier()

        base0 = wid * (n_chunks * K)

        def outer(oi, carry):
            for j in range(NBUF):
                base = base0 + (oi * NBUF + j) * K
                pltpu.sync_copy(src_hbm.at[pl.ds(base, K)], src_v.at[j])
                pltpu.sync_copy(dst_hbm.at[pl.ds(base, K)], dst_v.at[j])
                pltpu.sync_copy(w_hbm.at[pl.ds(base, K)], w_v.at[j])
                pltpu.sync_copy(h_hbm.at[src_v.at[j]], rows_v.at[j])

                def scale(g, carry2):
                    wv = w_v[j, pl.ds(g * 16, 16)]
                    for l in range(16):
                        we = wv[l]
                        e = g * 16 + l
                        for r in range(D // 16):
                            sl = pl.ds(r * 16, 16)
                            rows_v[j, e, sl] = rows_v[j, e, sl] * we
                    return carry2
                lax.fori_loop(0, K // 16, scale, 0)
                pltpu.sync_copy(rows_v.at[j], acc.at[dst_v.at[j]], add=True)
            return carry
        lax.fori_loop(0, n_chunks // NBUF, outer, 0)

        plsc.subcore_barrier()
        sl = pl.ds(s * RPT, RPT)
        pltpu.sync_copy(acc.at[sl], out_hbm.at[c, sl])

    return spmm(h, src, dst, w)


def _tc_mm(x, W):
    N, F = x.shape
    H = W.shape[1]
    BN = 1000

    def body(x_ref, w_ref, o_ref):
        o_ref[...] = jnp.dot(x_ref[...], w_ref[...],
                             preferred_element_type=jnp.float32)

    return pl.pallas_call(
        body,
        grid=(N // BN,),
        in_specs=[pl.BlockSpec((BN, F), lambda i: (i, 0)),
                  pl.BlockSpec((F, H), lambda i: (0, 0))],
        out_specs=pl.BlockSpec((BN, H), lambda i: (i, 0)),
        out_shape=jax.ShapeDtypeStruct((N, H), jnp.float32),
    )(x, W)


def _tc_relu(p0, p1, b):
    N, F = p0.shape
    BN = 1000

    def body(p0_ref, p1_ref, b_ref, o_ref):
        o_ref[...] = jnp.maximum(p0_ref[...] + p1_ref[...] + b_ref[...], 0.0)

    return pl.pallas_call(
        body,
        grid=(N // BN,),
        in_specs=[pl.BlockSpec((BN, F), lambda i: (i, 0)),
                  pl.BlockSpec((BN, F), lambda i: (i, 0)),
                  pl.BlockSpec((1, F), lambda i: (0, 0))],
        out_specs=pl.BlockSpec((BN, F), lambda i: (i, 0)),
        out_shape=jax.ShapeDtypeStruct((N, F), jnp.float32),
    )(p0, p1, b)


def _tc_out(q0, q1, W, b):
    N, F = q0.shape
    C = W.shape[1]
    BN = 1000

    def body(q0_ref, q1_ref, w_ref, b_ref, o_ref):
        o_ref[...] = jnp.dot(q0_ref[...] + q1_ref[...], w_ref[...],
                             preferred_element_type=jnp.float32) + b_ref[...]

    return pl.pallas_call(
        body,
        grid=(N // BN,),
        in_specs=[pl.BlockSpec((BN, F), lambda i: (i, 0)),
                  pl.BlockSpec((BN, F), lambda i: (i, 0)),
                  pl.BlockSpec((F, C), lambda i: (0, 0)),
                  pl.BlockSpec((1, C), lambda i: (0, 0))],
        out_specs=pl.BlockSpec((BN, C), lambda i: (i, 0)),
        out_shape=jax.ShapeDtypeStruct((N, C), jnp.float32),
    )(q0, q1, W, b)


def kernel(x, edge_index, edge_weight, W1, b1, W2, b2):
    N, F = x.shape
    E = edge_weight.shape[0]
    src = edge_index[0]
    dst = edge_index[1]

    sweep = NC * NS * K
    n_chunks = -(-E // sweep)
    n_chunks += (-n_chunks) % NBUF
    pad = n_chunks * sweep - E
    src_p = jnp.pad(src, (0, pad))
    dst_p = jnp.pad(dst, (0, pad))
    w_p = jnp.pad(edge_weight, (0, pad))

    h = _tc_mm(x, W1)
    p = _sc_spmm(h, src_p, dst_p, w_p, n_chunks)
    h2 = _tc_relu(p[0, :N], p[1, :N], b1.reshape(1, -1))
    q = _sc_spmm(h2, src_p, dst_p, w_p, n_chunks)
    return _tc_out(q[0, :N], q[1, :N], W2, b2.reshape(1, -1))


# speedup vs baseline: 3.3084x; 1.3536x over previous
import functools

import jax
import jax.numpy as jnp
from jax import lax
from jax.experimental import pallas as pl
from jax.experimental.pallas import tpu as pltpu
from jax.experimental.pallas import tpu_sc as plsc

NC = 2
NS = 16
K = 128


NBUF = 2
ZR = 32


def _sc_spmm(h, src, dst, w, n_chunks):
    N, D = h.shape
    NP = ((N + NS * ZR - 1) // (NS * ZR)) * NS * ZR
    RPT = NP // NS
    assert n_chunks % NBUF == 0 and RPT % ZR == 0
    mesh = plsc.VectorSubcoreMesh(core_axis_name="c", subcore_axis_name="s")

    @functools.partial(
        pl.kernel,
        mesh=mesh,
        out_type=jax.ShapeDtypeStruct((NC, NP, D), jnp.float32),
        scratch_types=[
            pltpu.VMEM((NBUF, K), jnp.int32),
            pltpu.VMEM((NBUF, K), jnp.int32),
            pltpu.VMEM((NBUF, K), jnp.float32),
            pltpu.VMEM((NBUF, K, D), jnp.float32),
            pltpu.VMEM((ZR, D), jnp.float32),
            pltpu.VMEM_SHARED((NP, D), jnp.float32),
            pltpu.SemaphoreType.DMA((NBUF,)),
            pltpu.SemaphoreType.DMA((NBUF,)),
        ],
    )
    def spmm(h_hbm, src_hbm, dst_hbm, w_hbm, out_hbm,
             src_v, dst_v, w_v, rows_v, zbuf, acc, isem, gsem):
        c = lax.axis_index("c")
        s = lax.axis_index("s")
        wid = c * NS + s

        def zrow(i, carry):
            for r in range(D // 16):
                zbuf[i, pl.ds(r * 16, 16)] = jnp.zeros((16,), jnp.float32)
            return carry
        lax.fori_loop(0, ZR, zrow, 0)
        for b in range(RPT // ZR):
            pltpu.sync_copy(zbuf, acc.at[pl.ds(s * RPT + b * ZR, ZR)])
        plsc.subcore_barrier()

        base0 = wid * (n_chunks * K)

        def start_idx(ci, j):
            base = base0 + ci * K
            pltpu.async_copy(src_hbm.at[pl.ds(base, K)], src_v.at[j],
                             isem.at[j])
            pltpu.async_copy(dst_hbm.at[pl.ds(base, K)], dst_v.at[j],
                             isem.at[j])
            pltpu.async_copy(w_hbm.at[pl.ds(base, K)], w_v.at[j], isem.at[j])

        def wait_idx(j):
            pltpu.make_async_copy(src_hbm.at[pl.ds(0, K)], src_v.at[j],
                                  isem.at[j]).wait()
            pltpu.make_async_copy(dst_hbm.at[pl.ds(0, K)], dst_v.at[j],
                                  isem.at[j]).wait()
            pltpu.make_async_copy(w_hbm.at[pl.ds(0, K)], w_v.at[j],
                                  isem.at[j]).wait()

        def start_gather(j):
            pltpu.async_copy(h_hbm.at[src_v.at[j]], rows_v.at[j], gsem.at[j])

        def wait_gather(j):
            pltpu.make_async_copy(h_hbm.at[src_v.at[j]], rows_v.at[j],
                                  gsem.at[j]).wait()

        start_idx(0, 0)
        wait_idx(0)
        start_gather(0)
        if n_chunks > 1:
            start_idx(1, 1)

        def outer(oi, carry):
            for j in range(NBUF):
                ci = oi * NBUF + j
                nj = 1 - j
                wait_gather(j)

                @pl.when(ci + 1 < n_chunks)
                def _():
                    wait_idx(nj)
                    start_gather(nj)

                def scale(g, carry2):
                    wv = w_v[j, pl.ds(g * 16, 16)]
                    for l in range(16):
                        we = wv[l]
                        e = g * 16 + l
                        for r in range(D // 16):
                            sl = pl.ds(r * 16, 16)
                            rows_v[j, e, sl] = rows_v[j, e, sl] * we
                    return carry2
                lax.fori_loop(0, K // 16, scale, 0)
                pltpu.sync_copy(rows_v.at[j], acc.at[dst_v.at[j]], add=True)

                @pl.when(ci + 2 < n_chunks)
                def _():
                    start_idx(ci + 2, j)
            return carry
        lax.fori_loop(0, n_chunks // NBUF, outer, 0)

        plsc.subcore_barrier()
        sl = pl.ds(s * RPT, RPT)
        pltpu.sync_copy(acc.at[sl], out_hbm.at[c, sl])

    return spmm(h, src, dst, w)


def _tc_mm(x, W):
    N, F = x.shape
    H = W.shape[1]
    BN = 1000

    def body(x_ref, w_ref, o_ref):
        o_ref[...] = jnp.dot(x_ref[...], w_ref[...],
                             preferred_element_type=jnp.float32)

    return pl.pallas_call(
        body,
        grid=(N // BN,),
        in_specs=[pl.BlockSpec((BN, F), lambda i: (i, 0)),
                  pl.BlockSpec((F, H), lambda i: (0, 0))],
        out_specs=pl.BlockSpec((BN, H), lambda i: (i, 0)),
        out_shape=jax.ShapeDtypeStruct((N, H), jnp.float32),
    )(x, W)


def _tc_relu(p0, p1, b):
    N, F = p0.shape
    BN = 1000

    def body(p0_ref, p1_ref, b_ref, o_ref):
        o_ref[...] = jnp.maximum(p0_ref[...] + p1_ref[...] + b_ref[...], 0.0)

    return pl.pallas_call(
        body,
        grid=(N // BN,),
        in_specs=[pl.BlockSpec((BN, F), lambda i: (i, 0)),
                  pl.BlockSpec((BN, F), lambda i: (i, 0)),
                  pl.BlockSpec((1, F), lambda i: (0, 0))],
        out_specs=pl.BlockSpec((BN, F), lambda i: (i, 0)),
        out_shape=jax.ShapeDtypeStruct((N, F), jnp.float32),
    )(p0, p1, b)


def _tc_out(q0, q1, W, b):
    N, F = q0.shape
    C = W.shape[1]
    BN = 1000

    def body(q0_ref, q1_ref, w_ref, b_ref, o_ref):
        o_ref[...] = jnp.dot(q0_ref[...] + q1_ref[...], w_ref[...],
                             preferred_element_type=jnp.float32) + b_ref[...]

    return pl.pallas_call(
        body,
        grid=(N // BN,),
        in_specs=[pl.BlockSpec((BN, F), lambda i: (i, 0)),
                  pl.BlockSpec((BN, F), lambda i: (i, 0)),
                  pl.BlockSpec((F, C), lambda i: (0, 0)),
                  pl.BlockSpec((1, C), lambda i: (0, 0))],
        out_specs=pl.BlockSpec((BN, C), lambda i: (i, 0)),
        out_shape=jax.ShapeDtypeStruct((N, C), jnp.float32),
    )(q0, q1, W, b)


def kernel(x, edge_index, edge_weight, W1, b1, W2, b2):
    N, F = x.shape
    E = edge_weight.shape[0]
    src = edge_index[0]
    dst = edge_index[1]

    sweep = NC * NS * K
    n_chunks = -(-E // sweep)
    n_chunks += (-n_chunks) % NBUF
    pad = n_chunks * sweep - E
    src_p = jnp.pad(src, (0, pad))
    dst_p = jnp.pad(dst, (0, pad))
    w_p = jnp.pad(edge_weight, (0, pad))

    h = _tc_mm(x, W1)
    p = _sc_spmm(h, src_p, dst_p, w_p, n_chunks)
    h2 = _tc_relu(p[0, :N], p[1, :N], b1.reshape(1, -1))
    q = _sc_spmm(h2, src_p, dst_p, w_p, n_chunks)
    return _tc_out(q[0, :N], q[1, :N], W2, b2.reshape(1, -1))


# speedup vs baseline: 3.3483x; 1.0121x over previous
import functools

import jax
import jax.numpy as jnp
from jax import lax
from jax.experimental import pallas as pl
from jax.experimental.pallas import tpu as pltpu
from jax.experimental.pallas import tpu_sc as plsc

NC = 2
NS = 16
K = 128


NBUF = 2
NDST = 4
ZR = 32


def _sc_spmm(h, src, dst, w, n_chunks):
    N, D = h.shape
    NP = ((N + NS * ZR - 1) // (NS * ZR)) * NS * ZR
    RPT = NP // NS
    assert n_chunks % NBUF == 0 and RPT % ZR == 0
    mesh = plsc.VectorSubcoreMesh(core_axis_name="c", subcore_axis_name="s")

    @functools.partial(
        pl.kernel,
        mesh=mesh,
        out_type=jax.ShapeDtypeStruct((NC, NP, D), jnp.float32),
        scratch_types=[
            pltpu.VMEM((NBUF, K), jnp.int32),
            pltpu.VMEM((NDST, K), jnp.int32),
            pltpu.VMEM((NBUF, K), jnp.float32),
            pltpu.VMEM((NBUF, K, D), jnp.float32),
            pltpu.VMEM((ZR, D), jnp.float32),
            pltpu.VMEM_SHARED((NP, D), jnp.float32),
            pltpu.SemaphoreType.DMA((NBUF,)),
            pltpu.SemaphoreType.DMA((NBUF,)),
            pltpu.SemaphoreType.DMA((NBUF,)),
        ],
    )
    def spmm(h_hbm, src_hbm, dst_hbm, w_hbm, out_hbm,
             src_v, dst_v, w_v, rows_v, zbuf, acc, isem, gsem, ssem):
        c = lax.axis_index("c")
        s = lax.axis_index("s")
        wid = c * NS + s

        def zrow(i, carry):
            for r in range(D // 16):
                zbuf[i, pl.ds(r * 16, 16)] = jnp.zeros((16,), jnp.float32)
            return carry
        lax.fori_loop(0, ZR, zrow, 0)
        for b in range(RPT // ZR):
            pltpu.sync_copy(zbuf, acc.at[pl.ds(s * RPT + b * ZR, ZR)])
        plsc.subcore_barrier()

        base0 = wid * (n_chunks * K)

        def start_idx(ci, j):
            base = base0 + ci * K
            pltpu.async_copy(src_hbm.at[pl.ds(base, K)], src_v.at[j],
                             isem.at[j])
            pltpu.async_copy(dst_hbm.at[pl.ds(base, K)],
                             dst_v.at[ci % NDST], isem.at[j])
            pltpu.async_copy(w_hbm.at[pl.ds(base, K)], w_v.at[j], isem.at[j])

        def wait_idx(j):
            pltpu.make_async_copy(src_hbm.at[pl.ds(0, K)], src_v.at[j],
                                  isem.at[j]).wait()
            pltpu.make_async_copy(dst_hbm.at[pl.ds(0, K)], dst_v.at[0],
                                  isem.at[j]).wait()
            pltpu.make_async_copy(w_hbm.at[pl.ds(0, K)], w_v.at[j],
                                  isem.at[j]).wait()

        def start_gather(j):
            pltpu.async_copy(h_hbm.at[src_v.at[j]], rows_v.at[j], gsem.at[j])

        def wait_gather(j):
            pltpu.make_async_copy(h_hbm.at[src_v.at[j]], rows_v.at[j],
                                  gsem.at[j]).wait()

        def start_scatter(j, ci):
            pltpu.async_copy(rows_v.at[j], acc.at[dst_v.at[ci % NDST]],
                             ssem.at[j], add=True)

        def wait_scatter(j):
            pltpu.make_async_copy(rows_v.at[j], acc.at[dst_v.at[0]],
                                  ssem.at[j]).wait()

        start_idx(0, 0)
        wait_idx(0)
        start_gather(0)
        if n_chunks > 1:
            start_idx(1, 1)

        def outer(oi, carry):
            for j in range(NBUF):
                ci = oi * NBUF + j
                nj = 1 - j
                wait_gather(j)

                @pl.when(ci + 1 < n_chunks)
                def _():
                    wait_idx(nj)

                    @pl.when(ci >= 1)
                    def _():
                        wait_scatter(nj)
                    start_gather(nj)

                def scale(g, carry2):
                    wv = w_v[j, pl.ds(g * 16, 16)]
                    for l in range(16):
                        we = wv[l]
                        e = g * 16 + l
                        for r in range(D // 16):
                            sl = pl.ds(r * 16, 16)
                            rows_v[j, e, sl] = rows_v[j, e, sl] * we
                    return carry2
                lax.fori_loop(0, K // 16, scale, 0)
                start_scatter(j, ci)

                @pl.when(ci + 2 < n_chunks)
                def _():
                    start_idx(ci + 2, j)
            return carry
        lax.fori_loop(0, n_chunks // NBUF, outer, 0)

        wait_scatter(0)
        if n_chunks > 1:
            wait_scatter(1)

        plsc.subcore_barrier()
        sl = pl.ds(s * RPT, RPT)
        pltpu.sync_copy(acc.at[sl], out_hbm.at[c, sl])

    return spmm(h, src, dst, w)


def _tc_mm(x, W):
    N, F = x.shape
    H = W.shape[1]
    BN = 1000

    def body(x_ref, w_ref, o_ref):
        o_ref[...] = jnp.dot(x_ref[...], w_ref[...],
                             preferred_element_type=jnp.float32)

    return pl.pallas_call(
        body,
        grid=(N // BN,),
        in_specs=[pl.BlockSpec((BN, F), lambda i: (i, 0)),
                  pl.BlockSpec((F, H), lambda i: (0, 0))],
        out_specs=pl.BlockSpec((BN, H), lambda i: (i, 0)),
        out_shape=jax.ShapeDtypeStruct((N, H), jnp.float32),
    )(x, W)


def _tc_relu(p0, p1, b):
    N, F = p0.shape
    BN = 1000

    def body(p0_ref, p1_ref, b_ref, o_ref):
        o_ref[...] = jnp.maximum(p0_ref[...] + p1_ref[...] + b_ref[...], 0.0)

    return pl.pallas_call(
        body,
        grid=(N // BN,),
        in_specs=[pl.BlockSpec((BN, F), lambda i: (i, 0)),
                  pl.BlockSpec((BN, F), lambda i: (i, 0)),
                  pl.BlockSpec((1, F), lambda i: (0, 0))],
        out_specs=pl.BlockSpec((BN, F), lambda i: (i, 0)),
        out_shape=jax.ShapeDtypeStruct((N, F), jnp.float32),
    )(p0, p1, b)


def _tc_out(q0, q1, W, b):
    N, F = q0.shape
    C = W.shape[1]
    BN = 1000

    def body(q0_ref, q1_ref, w_ref, b_ref, o_ref):
        o_ref[...] = jnp.dot(q0_ref[...] + q1_ref[...], w_ref[...],
                             preferred_element_type=jnp.float32) + b_ref[...]

    return pl.pallas_call(
        body,
        grid=(N // BN,),
        in_specs=[pl.BlockSpec((BN, F), lambda i: (i, 0)),
                  pl.BlockSpec((BN, F), lambda i: (i, 0)),
                  pl.BlockSpec((F, C), lambda i: (0, 0)),
                  pl.BlockSpec((1, C), lambda i: (0, 0))],
        out_specs=pl.BlockSpec((BN, C), lambda i: (i, 0)),
        out_shape=jax.ShapeDtypeStruct((N, C), jnp.float32),
    )(q0, q1, W, b)


def kernel(x, edge_index, edge_weight, W1, b1, W2, b2):
    N, F = x.shape
    E = edge_weight.shape[0]
    src = edge_index[0]
    dst = edge_index[1]

    sweep = NC * NS * K
    n_chunks = -(-E // sweep)
    n_chunks += (-n_chunks) % NBUF
    pad = n_chunks * sweep - E
    src_p = jnp.pad(src, (0, pad))
    dst_p = jnp.pad(dst, (0, pad))
    w_p = jnp.pad(edge_weight, (0, pad))

    h = _tc_mm(x, W1)
    p = _sc_spmm(h, src_p, dst_p, w_p, n_chunks)
    h2 = _tc_relu(p[0, :N], p[1, :N], b1.reshape(1, -1))
    q = _sc_spmm(h2, src_p, dst_p, w_p, n_chunks)
    return _tc_out(q[0, :N], q[1, :N], W2, b2.reshape(1, -1))


# speedup vs baseline: 9.5034x; 2.8383x over previous
import functools

import jax
import jax.numpy as jnp
from jax import lax
from jax.experimental import pallas as pl
from jax.experimental.pallas import tpu as pltpu
from jax.experimental.pallas import tpu_sc as plsc

NC = 2
NS = 16
K = 128


NBUF = 2
NDST = 4
ZR = 32


def _sc_spmm(h, src, dst, w, n_chunks):
    N, D = h.shape
    NP = ((N + NS * ZR - 1) // (NS * ZR)) * NS * ZR
    RPT = NP // NS
    assert n_chunks % NBUF == 0 and RPT % ZR == 0
    mesh = plsc.VectorSubcoreMesh(core_axis_name="c", subcore_axis_name="s")

    @functools.partial(
        pl.kernel,
        mesh=mesh,
        out_type=jax.ShapeDtypeStruct((NC, NP, D), jnp.float32),
        scratch_types=[
            pltpu.VMEM((NBUF, K), jnp.int32),
            pltpu.VMEM((NDST, K), jnp.int32),
            pltpu.VMEM((NBUF, K), jnp.float32),
            pltpu.VMEM((NBUF, K, D), jnp.float32),
            pltpu.VMEM((ZR, D), jnp.float32),
            pltpu.VMEM_SHARED((NP, D), jnp.float32),
            pltpu.SemaphoreType.DMA((NBUF,)),
            pltpu.SemaphoreType.DMA((NBUF,)),
            pltpu.SemaphoreType.DMA((NBUF,)),
        ],
    )
    def spmm(h_hbm, src_hbm, dst_hbm, w_hbm, out_hbm,
             src_v, dst_v, w_v, rows_v, zbuf, acc, isem, gsem, ssem):
        c = lax.axis_index("c")
        s = lax.axis_index("s")
        wid = c * NS + s

        def zrow(i, carry):
            for r in range(D // 16):
                zbuf[i, pl.ds(r * 16, 16)] = jnp.zeros((16,), jnp.float32)
            return carry
        lax.fori_loop(0, ZR, zrow, 0)
        for b in range(RPT // ZR):
            pltpu.sync_copy(zbuf, acc.at[pl.ds(s * RPT + b * ZR, ZR)])
        plsc.subcore_barrier()

        base0 = wid * (n_chunks * K)

        def start_idx(ci, j):
            base = base0 + ci * K
            pltpu.async_copy(src_hbm.at[pl.ds(base, K)], src_v.at[j],
                             isem.at[j])
            pltpu.async_copy(dst_hbm.at[pl.ds(base, K)],
                             dst_v.at[ci % NDST], isem.at[j])
            pltpu.async_copy(w_hbm.at[pl.ds(base, K)], w_v.at[j], isem.at[j])

        def wait_idx(j):
            pltpu.make_async_copy(src_hbm.at[pl.ds(0, K)], src_v.at[j],
                                  isem.at[j]).wait()
            pltpu.make_async_copy(dst_hbm.at[pl.ds(0, K)], dst_v.at[0],
                                  isem.at[j]).wait()
            pltpu.make_async_copy(w_hbm.at[pl.ds(0, K)], w_v.at[j],
                                  isem.at[j]).wait()

        def start_gather(j):
            pltpu.async_copy(h_hbm.at[src_v.at[j]], rows_v.at[j], gsem.at[j])

        def wait_gather(j):
            pltpu.make_async_copy(h_hbm.at[src_v.at[j]], rows_v.at[j],
                                  gsem.at[j]).wait()

        def start_scatter(j, ci):
            pltpu.async_copy(rows_v.at[j], acc.at[dst_v.at[ci % NDST]],
                             ssem.at[j], add=True)

        def wait_scatter(j):
            pltpu.make_async_copy(rows_v.at[j], acc.at[dst_v.at[0]],
                                  ssem.at[j]).wait()

        start_idx(0, 0)
        wait_idx(0)
        start_gather(0)
        if n_chunks > 1:
            start_idx(1, 1)

        def outer(oi, carry):
            for j in range(NBUF):
                ci = oi * NBUF + j
                nj = 1 - j
                wait_gather(j)

                @pl.when(ci + 1 < n_chunks)
                def _():
                    wait_idx(nj)

                    @pl.when(ci >= 1)
                    def _():
                        wait_scatter(nj)
                    start_gather(nj)

                def scale(g, carry2):
                    wv = w_v[j, pl.ds(g * 16, 16)]
                    for l in range(16):
                        we = wv[l]
                        e = g * 16 + l
                        for r in range(D // 16):
                            sl = pl.ds(r * 16, 16)
                            rows_v[j, e, sl] = rows_v[j, e, sl] * we
                    return carry2
                lax.fori_loop(0, K // 16, scale, 0)
                start_scatter(j, ci)

                @pl.when(ci + 2 < n_chunks)
                def _():
                    start_idx(ci + 2, j)
            return carry
        lax.fori_loop(0, n_chunks // NBUF, outer, 0)

        wait_scatter(0)
        if n_chunks > 1:
            wait_scatter(1)

        plsc.subcore_barrier()
        sl = pl.ds(s * RPT, RPT)
        pltpu.sync_copy(acc.at[sl], out_hbm.at[c, sl])

    return spmm(h, src, dst, w)


def _tc_mm(x, W):
    N, F = x.shape
    H = W.shape[1]
    BN = 1000

    def body(x_ref, w_ref, o_ref):
        o_ref[...] = jnp.dot(x_ref[...], w_ref[...],
                             preferred_element_type=jnp.float32)

    return pl.pallas_call(
        body,
        grid=(N // BN,),
        in_specs=[pl.BlockSpec((BN, F), lambda i: (i, 0)),
                  pl.BlockSpec((F, H), lambda i: (0, 0))],
        out_specs=pl.BlockSpec((BN, H), lambda i: (i, 0)),
        out_shape=jax.ShapeDtypeStruct((N, H), jnp.float32),
    )(x, W)


def _tc_relu(p0, p1, b):
    N, F = p0.shape
    BN = 1000

    def body(p0_ref, p1_ref, b_ref, o_ref):
        o_ref[...] = jnp.maximum(p0_ref[...] + p1_ref[...] + b_ref[...], 0.0)

    return pl.pallas_call(
        body,
        grid=(N // BN,),
        in_specs=[pl.BlockSpec((BN, F), lambda i: (i, 0)),
                  pl.BlockSpec((BN, F), lambda i: (i, 0)),
                  pl.BlockSpec((1, F), lambda i: (0, 0))],
        out_specs=pl.BlockSpec((BN, F), lambda i: (i, 0)),
        out_shape=jax.ShapeDtypeStruct((N, F), jnp.float32),
    )(p0, p1, b)


def _tc_out(q0, q1, W, b):
    N, F = q0.shape
    C = W.shape[1]
    BN = 1000

    def body(q0_ref, q1_ref, w_ref, b_ref, o_ref):
        o_ref[...] = jnp.dot(q0_ref[...] + q1_ref[...], w_ref[...],
                             preferred_element_type=jnp.float32) + b_ref[...]

    return pl.pallas_call(
        body,
        grid=(N // BN,),
        in_specs=[pl.BlockSpec((BN, F), lambda i: (i, 0)),
                  pl.BlockSpec((BN, F), lambda i: (i, 0)),
                  pl.BlockSpec((F, C), lambda i: (0, 0)),
                  pl.BlockSpec((1, C), lambda i: (0, 0))],
        out_specs=pl.BlockSpec((BN, C), lambda i: (i, 0)),
        out_shape=jax.ShapeDtypeStruct((N, C), jnp.float32),
    )(q0, q1, W, b)


def kernel(x, edge_index, edge_weight, W1, b1, W2, b2):
    N, F = x.shape
    E = edge_weight.shape[0]
    src = edge_index[0]
    dst = edge_index[1]

    sweep = NC * NS * K
    n_chunks = -(-E // sweep)
    n_chunks += (-n_chunks) % NBUF
    pad = n_chunks * sweep - E
    fill = (jnp.arange(pad, dtype=jnp.int32) * 8) % N
    src_p = jnp.concatenate([src, fill])
    dst_p = jnp.concatenate([dst, fill])
    w_p = jnp.pad(edge_weight, (0, pad))

    h = _tc_mm(x, W1)
    p = _sc_spmm(h, src_p, dst_p, w_p, n_chunks)
    h2 = _tc_relu(p[0, :N], p[1, :N], b1.reshape(1, -1))
    q = _sc_spmm(h2, src_p, dst_p, w_p, n_chunks)
    return _tc_out(q[0, :N], q[1, :N], W2, b2.reshape(1, -1))


# speedup vs baseline: 9.8068x; 1.0319x over previous
import functools

import jax
import jax.numpy as jnp
from jax import lax
from jax.experimental import pallas as pl
from jax.experimental.pallas import tpu as pltpu
from jax.experimental.pallas import tpu_sc as plsc

NC = 2
NS = 16
K = 128


NBUF = 2
NDST = 4
ZR = 32


def _sc_spmm(h, src, dst, w, n_chunks):
    N, D = h.shape
    NP = ((N + NS * ZR - 1) // (NS * ZR)) * NS * ZR
    RPT = NP // NS
    assert n_chunks % NBUF == 0 and RPT % ZR == 0
    mesh = plsc.VectorSubcoreMesh(core_axis_name="c", subcore_axis_name="s")

    @functools.partial(
        pl.kernel,
        mesh=mesh,
        out_type=jax.ShapeDtypeStruct((NC, NP, D), jnp.float32),
        scratch_types=[
            pltpu.VMEM((NBUF, K), jnp.int32),
            pltpu.VMEM((NDST, K), jnp.int32),
            pltpu.VMEM((NBUF, K), jnp.float32),
            pltpu.VMEM((NBUF, K, D), jnp.float32),
            pltpu.VMEM((ZR, D), jnp.float32),
            pltpu.VMEM_SHARED((NP, D), jnp.float32),
            pltpu.SemaphoreType.DMA((NBUF,)),
            pltpu.SemaphoreType.DMA((NBUF,)),
            pltpu.SemaphoreType.DMA((NBUF,)),
        ],
    )
    def spmm(h_hbm, src_hbm, dst_hbm, w_hbm, out_hbm,
             src_v, dst_v, w_v, rows_v, zbuf, acc, isem, gsem, ssem):
        c = lax.axis_index("c")
        s = lax.axis_index("s")
        wid = c * NS + s

        def zrow(i, carry):
            for r in range(D // 16):
                zbuf[i, pl.ds(r * 16, 16)] = jnp.zeros((16,), jnp.float32)
            return carry
        lax.fori_loop(0, ZR, zrow, 0)
        for b in range(RPT // ZR):
            pltpu.sync_copy(zbuf, acc.at[pl.ds(s * RPT + b * ZR, ZR)])
        plsc.subcore_barrier()

        base0 = wid * (n_chunks * K)

        def start_idx(ci, j):
            base = base0 + ci * K
            pltpu.async_copy(src_hbm.at[pl.ds(base, K)], src_v.at[j],
                             isem.at[j])
            pltpu.async_copy(dst_hbm.at[pl.ds(base, K)],
                             dst_v.at[ci % NDST], isem.at[j])
            pltpu.async_copy(w_hbm.at[pl.ds(base, K)], w_v.at[j], isem.at[j])

        def wait_idx(j):
            pltpu.make_async_copy(src_hbm.at[pl.ds(0, K)], src_v.at[j],
                                  isem.at[j]).wait()
            pltpu.make_async_copy(dst_hbm.at[pl.ds(0, K)], dst_v.at[0],
                                  isem.at[j]).wait()
            pltpu.make_async_copy(w_hbm.at[pl.ds(0, K)], w_v.at[j],
                                  isem.at[j]).wait()

        def start_gather(j):
            pltpu.async_copy(h_hbm.at[src_v.at[j]], rows_v.at[j], gsem.at[j])

        def wait_gather(j):
            pltpu.make_async_copy(h_hbm.at[src_v.at[j]], rows_v.at[j],
                                  gsem.at[j]).wait()

        def start_scatter(j, ci):
            pltpu.async_copy(rows_v.at[j], acc.at[dst_v.at[ci % NDST]],
                             ssem.at[j], add=True)

        def wait_scatter(j):
            pltpu.make_async_copy(rows_v.at[j], acc.at[dst_v.at[0]],
                                  ssem.at[j]).wait()

        start_idx(0, 0)
        wait_idx(0)
        start_gather(0)
        if n_chunks > 1:
            start_idx(1, 1)

        def outer(oi, carry):
            for j in range(NBUF):
                ci = oi * NBUF + j
                nj = 1 - j
                wait_gather(j)

                @pl.when(ci + 1 < n_chunks)
                def _():
                    wait_idx(nj)

                    @pl.when(ci >= 1)
                    def _():
                        wait_scatter(nj)
                    start_gather(nj)

                def scale(g, carry2):
                    wv = w_v[j, pl.ds(g * 16, 16)]
                    for l in range(16):
                        we = wv[l]
                        e = g * 16 + l
                        for r in range(D // 16):
                            sl = pl.ds(r * 16, 16)
                            rows_v[j, e, sl] = rows_v[j, e, sl] * we
                    return carry2
                lax.fori_loop(0, K // 16, scale, 0, unroll=2)
                start_scatter(j, ci)

                @pl.when(ci + 2 < n_chunks)
                def _():
                    start_idx(ci + 2, j)
            return carry
        lax.fori_loop(0, n_chunks // NBUF, outer, 0)

        wait_scatter(0)
        if n_chunks > 1:
            wait_scatter(1)

        plsc.subcore_barrier()
        sl = pl.ds(s * RPT, RPT)
        pltpu.sync_copy(acc.at[sl], out_hbm.at[c, sl])

    return spmm(h, src, dst, w)


def _tc_mm(x, W):
    N, F = x.shape
    H = W.shape[1]
    BN = 1000

    def body(x_ref, w_ref, o_ref):
        o_ref[...] = jnp.dot(x_ref[...], w_ref[...],
                             preferred_element_type=jnp.float32)

    return pl.pallas_call(
        body,
        grid=(N // BN,),
        in_specs=[pl.BlockSpec((BN, F), lambda i: (i, 0)),
                  pl.BlockSpec((F, H), lambda i: (0, 0))],
        out_specs=pl.BlockSpec((BN, H), lambda i: (i, 0)),
        out_shape=jax.ShapeDtypeStruct((N, H), jnp.float32),
    )(x, W)


def _tc_relu(p, b):
    _, NPD, F = p.shape
    BN = 1024

    def body(p0_ref, p1_ref, b_ref, o_ref):
        o_ref[...] = jnp.maximum(p0_ref[0] + p1_ref[0] + b_ref[...], 0.0)

    return pl.pallas_call(
        body,
        grid=(NPD // BN,),
        in_specs=[pl.BlockSpec((1, BN, F), lambda i: (0, i, 0)),
                  pl.BlockSpec((1, BN, F), lambda i: (1, i, 0)),
                  pl.BlockSpec((1, F), lambda i: (0, 0))],
        out_specs=pl.BlockSpec((BN, F), lambda i: (i, 0)),
        out_shape=jax.ShapeDtypeStruct((NPD, F), jnp.float32),
    )(p, p, b)


def _tc_out(q, W, b):
    _, NPD, F = q.shape
    C = W.shape[1]
    BN = 1024

    def body(q0_ref, q1_ref, w_ref, b_ref, o_ref):
        o_ref[...] = jnp.dot(q0_ref[0] + q1_ref[0], w_ref[...],
                             preferred_element_type=jnp.float32) + b_ref[...]

    return pl.pallas_call(
        body,
        grid=(NPD // BN,),
        in_specs=[pl.BlockSpec((1, BN, F), lambda i: (0, i, 0)),
                  pl.BlockSpec((1, BN, F), lambda i: (1, i, 0)),
                  pl.BlockSpec((F, C), lambda i: (0, 0)),
                  pl.BlockSpec((1, C), lambda i: (0, 0))],
        out_specs=pl.BlockSpec((BN, C), lambda i: (i, 0)),
        out_shape=jax.ShapeDtypeStruct((NPD, C), jnp.float32),
    )(q, q, W, b)


def kernel(x, edge_index, edge_weight, W1, b1, W2, b2):
    N, F = x.shape
    E = edge_weight.shape[0]
    src = edge_index[0]
    dst = edge_index[1]

    sweep = NC * NS * K
    n_chunks = -(-E // sweep)
    n_chunks += (-n_chunks) % NBUF
    pad = n_chunks * sweep - E
    fill = (jnp.arange(pad, dtype=jnp.int32) * 8) % N
    src_p = jnp.concatenate([src, fill])
    dst_p = jnp.concatenate([dst, fill])
    w_p = jnp.pad(edge_weight, (0, pad))

    h = _tc_mm(x, W1)
    p = _sc_spmm(h, src_p, dst_p, w_p, n_chunks)
    h2 = _tc_relu(p, b1.reshape(1, -1))
    q = _sc_spmm(h2, src_p, dst_p, w_p, n_chunks)
    return _tc_out(q, W2, b2.reshape(1, -1))[:N]
